# baseline (speedup 1.0000x reference)
import jax
import jax.numpy as jnp
from jax import lax
from jax.experimental import pallas as pl
from jax.experimental.pallas import tpu as pltpu
from jax.experimental.pallas import tpu_sc as plsc

_BATCH = 16384
_HIST = 50
_FEATURES = 64
_N = _BATCH * _HIST
_NC = 2
_NS = 16
_NW = _NC * _NS
_PER_W = _N // _NW
_CHUNK = 512
_NCHUNK = _PER_W // _CHUNK


def _embed_body(idx_hbm, table_hbm, out_hbm, idx_v, rows_v, sem):
    wid = lax.axis_index("s") * _NC + lax.axis_index("c")
    base = wid * _PER_W
    pltpu.sync_copy(idx_hbm.at[pl.ds(base, _PER_W)], idx_v)

    @pl.loop(0, _NCHUNK)
    def _(j):
        off = j * _CHUNK
        pltpu.async_copy(
            table_hbm.at[idx_v.at[pl.ds(off, _CHUNK)]], rows_v, sem
        ).wait()
        pltpu.sync_copy(rows_v, out_hbm.at[pl.ds(base + off, _CHUNK)])


@jax.jit
def kernel(inputs, embedding):
    idx = inputs.reshape(_N)
    out = pl.kernel(
        _embed_body,
        out_type=jax.ShapeDtypeStruct((_N, _FEATURES), jnp.float32),
        mesh=plsc.VectorSubcoreMesh(core_axis_name="c", subcore_axis_name="s"),
        compiler_params=pltpu.CompilerParams(use_tc_tiling_on_sc=False),
        scratch_types=[
            pltpu.VMEM((_PER_W,), jnp.int32),
            pltpu.VMEM((_CHUNK, _FEATURES), jnp.float32),
            pltpu.SemaphoreType.DMA,
        ],
    )(idx, embedding)
    return out.reshape(_BATCH, _HIST, _FEATURES)


# speedup vs baseline: 1.0245x; 1.0245x over previous
import jax
import jax.numpy as jnp
from jax import lax
from jax.experimental import pallas as pl
from jax.experimental.pallas import tpu as pltpu
from jax.experimental.pallas import tpu_sc as plsc

_BATCH = 16384
_HIST = 50
_FEATURES = 64
_N = _BATCH * _HIST
_NC = 2
_NS = 16
_NW = _NC * _NS
_PER_W = _N // _NW
_CHUNK = 400
_NCHUNK = _PER_W // _CHUNK
_NBUF = 3


def _embed_body(idx_hbm, table_hbm, out_hbm, idx_v, rows_v, gsem, osem):
    wid = lax.axis_index("s") * _NC + lax.axis_index("c")
    base = wid * _PER_W
    pltpu.sync_copy(idx_hbm.at[pl.ds(base, _PER_W)], idx_v)

    def issue_gather(j, s):
        pltpu.async_copy(
            table_hbm.at[idx_v.at[pl.ds(j * _CHUNK, _CHUNK)]],
            rows_v.at[s], gsem.at[s])

    def wait_gather(s):
        pltpu.make_async_copy(
            table_hbm.at[pl.ds(0, _CHUNK)], rows_v.at[s], gsem.at[s]).wait()

    def issue_wb(j, s):
        pltpu.async_copy(
            rows_v.at[s], out_hbm.at[pl.ds(base + j * _CHUNK, _CHUNK)],
            osem.at[s])

    def wait_wb(s):
        pltpu.make_async_copy(
            rows_v.at[s], out_hbm.at[pl.ds(base, _CHUNK)], osem.at[s]).wait()

    for b in range(_NBUF - 1):
        issue_gather(b, b)

    wait_gather(0)
    issue_wb(0, 0)
    issue_gather(_NBUF - 1, _NBUF - 1)

    @pl.loop(1, _NCHUNK, step=_NBUF)
    def _(j0):
        for b in range(_NBUF):
            j = j0 + b
            s = (1 + b) % _NBUF
            sp = b % _NBUF
            wait_gather(s)
            issue_wb(j, s)

            @pl.when(j < _NCHUNK - _NBUF + 1)
            def _():
                wait_wb(sp)
                issue_gather(j - 1 + _NBUF, sp)

    for j in range(_NCHUNK - _NBUF, _NCHUNK):
        wait_wb(j % _NBUF)


@jax.jit
def kernel(inputs, embedding):
    idx = inputs.reshape(_N)
    out = pl.kernel(
        _embed_body,
        out_type=jax.ShapeDtypeStruct((_N, _FEATURES), jnp.float32),
        mesh=plsc.VectorSubcoreMesh(core_axis_name="c", subcore_axis_name="s"),
        compiler_params=pltpu.CompilerParams(use_tc_tiling_on_sc=False),
        scratch_types=[
            pltpu.VMEM((_PER_W,), jnp.int32),
            pltpu.VMEM((_NBUF, _CHUNK, _FEATURES), jnp.float32),
            pltpu.SemaphoreType.DMA((_NBUF,)),
            pltpu.SemaphoreType.DMA((_NBUF,)),
        ],
    )(idx, embedding)
    return out.reshape(_BATCH, _HIST, _FEATURES)
